# baseline (speedup 1.0000x reference)
import functools

import jax
import jax.numpy as jnp
from jax import lax
from jax.experimental import pallas as pl
from jax.experimental.pallas import tpu as pltpu
from jax.experimental.pallas import tpu_sc as plsc

_SEQ = 200
_BATCH = 4096
_DIM = 32
_NC = 2
_NS = 16
_NW = _NC * _NS
_BPW = _BATCH // _NW
_SBLK = 4
_NBLK = _SEQ // _SBLK
_NBUF = 2
_NT = _SBLK * 4
_PAD = _BPW + 1
_VOCAB = 1000000
_TCOLS = 7813
_TFULL = _TCOLS - 1


def _conv_body(tt_hbm, tail_hbm, out_hbm, stg, cbuf0, cbuf1, gsems, osems):
    c = lax.axis_index("c")
    s = lax.axis_index("s")
    wid = s * _NC + c
    base = wid * (_TFULL // _NW) + jnp.minimum(wid, _TFULL % _NW)
    cnt = (_TFULL // _NW) + jnp.where(wid < _TFULL % _NW, 1, 0)

    iota16 = lax.iota(jnp.int32, 16)
    rows0 = iota16
    rows1 = iota16 + 16

    def start_stage(t, b):
        for g in range(4):
            pltpu.async_copy(
                tt_hbm.at[pl.ds(g * 8, 8), pl.ds(t * _BPW, _BPW)],
                stg.at[b, pl.ds(g * 8, 8), pl.ds(0, _BPW)], gsems.at[b])

    def wait_stage(b):
        for g in range(4):
            pltpu.make_async_copy(
                tt_hbm.at[pl.ds(0, 8), pl.ds(0, _BPW)],
                stg.at[b, pl.ds(g * 8, 8), pl.ds(0, _BPW)], gsems.at[b]).wait()

    cbufs = (cbuf0, cbuf1)

    def start_out(t, b, nel):
        pltpu.async_copy(
            cbufs[b].at[pl.ds(0, nel)],
            out_hbm.at[pl.ds(t * 4096, nel)], osems.at[b])

    def wait_out(b, nel):
        pltpu.make_async_copy(
            cbufs[b].at[pl.ds(0, nel)],
            out_hbm.at[pl.ds(0, nel)], osems.at[b]).wait()

    def transpose(b):
        sg = stg.at[b]
        cb = cbufs[b]

        def rowfn(r, carry):
            colv = jnp.full((16,), r, jnp.int32)
            tgt = iota16 + r * _DIM
            plsc.store_scatter(cb, [tgt], plsc.load_gather(sg, [rows0, colv]))
            plsc.store_scatter(cb, [tgt + 16], plsc.load_gather(sg, [rows1, colv]))
            return carry

        lax.fori_loop(0, _BPW, rowfn, 0, unroll=False)

    nmax = (_TFULL // _NW) + 1

    @pl.when(cnt > 0)
    def _():
        start_stage(base, 0)

    def pair(p, carry):
        for b in range(2):
            i = p * 2 + b

            @pl.when(i < cnt)
            def _():
                @pl.when(i + 1 < cnt)
                def _():
                    start_stage(base + i + 1, 1 - b)

                wait_stage(b)

                @pl.when(i >= 2)
                def _():
                    wait_out(b, 4096)

                transpose(b)
                start_out(base + i, b, 4096)
        return carry

    lax.fori_loop(0, (nmax + 1) // 2, pair, 0, unroll=False)

    @pl.when(cnt > 0)
    def _():
        wait_out(0, 4096)

    @pl.when(cnt > 1)
    def _():
        wait_out(1, 4096)

    ntail = _VOCAB - _TFULL * _BPW

    @pl.when(wid == _NW - 1)
    def _():
        for g in range(4):
            pltpu.async_copy(
                tail_hbm.at[pl.ds(g * 8, 8), pl.ds(0, _BPW)],
                stg.at[0, pl.ds(g * 8, 8), pl.ds(0, _BPW)], gsems.at[0])
        wait_stage(0)
        transpose(0)
        start_out(_TFULL, 0, ntail * _DIM)
        wait_out(0, ntail * _DIM)


def _body(idx_hbm, tok_hbm, pos_hbm, out_hbm, idx_v, pos_v, gbuf, obuf, gsems, osems):
    c = lax.axis_index("c")
    s = lax.axis_index("s")
    wid = s * _NC + c

    pltpu.sync_copy(idx_hbm.at[pl.ds(0, _SEQ // 8), wid], idx_v)
    pltpu.sync_copy(pos_hbm, pos_v)

    iota16 = lax.iota(jnp.int32, 16)


    def start_gathers(blk, b):
        for sl in range(_SBLK):
            sq = blk * _SBLK + sl
            pltpu.async_copy(
                tok_hbm.at[idx_v.at[sq // 8, lax.rem(sq, 8)]],
                gbuf.at[b, pl.ds(sl * _BPW, _BPW)], gsems.at[b])

    def wait_gathers(b):
        pltpu.make_async_copy(
            tok_hbm.at[pl.ds(0, _SBLK * _BPW)], gbuf.at[b], gsems.at[b]).wait()

    def start_out(blk, b):
        for t in range(_NT):
            sl, tr = t // 4, t % 4
            pltpu.async_copy(
                obuf.at[b, pl.ds(t * 8, 8), pl.ds(0, _BPW)],
                out_hbm.at[blk * _SBLK + sl, tr, wid], osems.at[b])

    def wait_out(b):
        for t in range(_NT):
            pltpu.make_async_copy(
                obuf.at[b, pl.ds(t * 8, 8), pl.ds(0, _BPW)],
                out_hbm.at[0, 0, 0], osems.at[b]).wait()

    def transpose_add(blk, b):
        s0 = blk * _SBLK
        ob = obuf.at[b]
        for sl in range(_SBLK):
            pbase = (s0 + sl) * _DIM
            p0 = pos_v[pl.ds(pbase, 16)]
            p1 = pos_v[pl.ds(pbase + 16, 16)]
            rows0 = iota16 + (sl * _DIM)
            rows1 = rows0 + 16

            def rowfn(r, carry):
                g = sl * _BPW + r
                colv = jnp.full((16,), r, jnp.int32)
                a0 = gbuf[b, g, pl.ds(0, 16)] + p0
                a1 = gbuf[b, g, pl.ds(16, 16)] + p1
                plsc.store_scatter(ob, [rows0, colv], a0)
                plsc.store_scatter(ob, [rows1, colv], a1)
                return carry

            lax.fori_loop(0, _BPW, rowfn, 0, unroll=False)

    def slot(blk, b, first, last):
        if not last:
            start_gathers(blk + 1, 1 - b)
        wait_gathers(b)
        if not first:
            wait_out(b)
        transpose_add(blk, b)
        start_out(blk, b)

    start_gathers(0, 0)
    slot(0, 0, first=True, last=False)
    slot(1, 1, first=True, last=False)

    def group(g, carry):
        slot(g * 2, 0, first=False, last=False)
        slot(g * 2 + 1, 1, first=False, last=False)
        return carry

    lax.fori_loop(1, _NBLK // 2 - 1, group, 0, unroll=False)

    slot(_NBLK - 2, 0, first=False, last=False)
    slot(_NBLK - 1, 1, first=False, last=True)

    wait_out(0)
    wait_out(1)


@jax.jit
def kernel(inputs, token_table, pos_table):
    idx = inputs.T.reshape(_SEQ // 8, 8, _NW, _BPW).transpose(0, 2, 1, 3)
    pos = pos_table.reshape(-1)
    conv = pl.kernel(
        _conv_body,
        out_type=jax.ShapeDtypeStruct((_VOCAB * _DIM,), jnp.float32),
        mesh=plsc.VectorSubcoreMesh(core_axis_name="c", subcore_axis_name="s"),
        compiler_params=pltpu.CompilerParams(
            use_tc_tiling_on_sc=True, needs_layout_passes=False),
        scratch_types=[
            pltpu.VMEM((2, 32, 136), jnp.float32),
            pltpu.VMEM((4096,), jnp.float32),
            pltpu.VMEM((4096,), jnp.float32),
            pltpu.SemaphoreType.DMA((2,)),
            pltpu.SemaphoreType.DMA((2,)),
        ],
    )
    tailp = jnp.pad(token_table[_TFULL * _BPW:].T, ((0, 0), (0, _BPW - (_VOCAB - _TFULL * _BPW))))
    tok = conv(token_table.T, tailp).reshape(-1, _DIM)
    run = pl.kernel(
        _body,
        out_type=jax.ShapeDtypeStruct((_SEQ, _DIM // 8, _NW, 8, _BPW), jnp.float32),
        mesh=plsc.VectorSubcoreMesh(core_axis_name="c", subcore_axis_name="s"),
        compiler_params=pltpu.CompilerParams(
            use_tc_tiling_on_sc=False, needs_layout_passes=False),
        scratch_types=[
            pltpu.VMEM((_SEQ // 8, 8, _BPW), jnp.int32),
            pltpu.VMEM((_SEQ * _DIM,), jnp.float32),
            pltpu.VMEM((_NBUF, _SBLK * _BPW, _DIM), jnp.float32),
            pltpu.VMEM((_NBUF, _NT * 8, _PAD), jnp.float32),
            pltpu.SemaphoreType.DMA((_NBUF,)),
            pltpu.SemaphoreType.DMA((_NBUF,)),
        ],
    )
    out5 = run(idx, tok, pos)
    return out5.transpose(2, 4, 0, 1, 3).reshape(_BATCH, _SEQ, _DIM)


# speedup vs baseline: 1.4356x; 1.4356x over previous
import functools

import jax
import jax.numpy as jnp
from jax import lax
from jax.experimental import pallas as pl
from jax.experimental.pallas import tpu as pltpu
from jax.experimental.pallas import tpu_sc as plsc

_SEQ = 200
_BATCH = 4096
_DIM = 32
_NC = 2
_NS = 16
_NW = _NC * _NS
_BPW = _BATCH // _NW
_SBLK = 4
_NBLK = _SEQ // _SBLK
_NBUF = 2
_NT = _SBLK * 4
_PAD = _BPW + 1


def _body(idx_hbm, tok_hbm, pos_hbm, out_hbm, idx_v, pos_v, gbuf, obuf, gsems, osems):
    c = lax.axis_index("c")
    s = lax.axis_index("s")
    wid = s * _NC + c

    pltpu.sync_copy(idx_hbm.at[pl.ds(0, _SEQ // 8), wid], idx_v)
    pltpu.sync_copy(pos_hbm, pos_v)

    iota16 = lax.iota(jnp.int32, 16)

    def scale_idx(i, carry):
        t = i // 8
        k = lax.rem(i, 8)
        for j in range(8):
            sl = pl.ds(j * 16, 16)
            idx_v[t, k, sl] = idx_v[t, k, sl] * 4
        return carry

    lax.fori_loop(0, _SEQ, scale_idx, 0, unroll=False)

    def start_gathers(blk, b):
        for sl in range(_SBLK):
            sq = blk * _SBLK + sl
            pltpu.async_copy(
                tok_hbm.at[idx_v.at[sq // 8, lax.rem(sq, 8)]],
                gbuf.at[b, pl.ds(sl * _BPW, _BPW)], gsems.at[b])

    def wait_gathers(b):
        pltpu.make_async_copy(
            tok_hbm.at[pl.ds(0, _SBLK * _BPW)], gbuf.at[b], gsems.at[b]).wait()

    def start_out(blk, b):
        for t in range(_NT):
            sl, tr = t // 4, t % 4
            pltpu.async_copy(
                obuf.at[b, pl.ds(t * 8, 8), pl.ds(0, _BPW)],
                out_hbm.at[blk * _SBLK + sl, tr, wid], osems.at[b])

    def wait_out(b):
        for t in range(_NT):
            pltpu.make_async_copy(
                obuf.at[b, pl.ds(t * 8, 8), pl.ds(0, _BPW)],
                out_hbm.at[0, 0, 0], osems.at[b]).wait()

    def transpose_add(blk, b):
        s0 = blk * _SBLK
        ob = obuf.at[b]
        for sl in range(_SBLK):
            pbase = (s0 + sl) * _DIM
            p0 = pos_v[pl.ds(pbase, 16)]
            p1 = pos_v[pl.ds(pbase + 16, 16)]
            rows0 = iota16 + (sl * _DIM)
            rows1 = rows0 + 16

            def rowfn(r, carry):
                g = sl * _BPW + r
                colv = jnp.full((16,), r, jnp.int32)
                a0 = gbuf[b, g, pl.ds(0, 16)] + p0
                a1 = gbuf[b, g, pl.ds(16, 16)] + p1
                plsc.store_scatter(ob, [rows0, colv], a0)
                plsc.store_scatter(ob, [rows1, colv], a1)
                return carry

            lax.fori_loop(0, _BPW, rowfn, 0, unroll=False)

    def slot(blk, b, first, last):
        if not last:
            start_gathers(blk + 1, 1 - b)
        wait_gathers(b)
        if not first:
            wait_out(b)
        transpose_add(blk, b)
        start_out(blk, b)

    start_gathers(0, 0)
    slot(0, 0, first=True, last=False)
    slot(1, 1, first=True, last=False)

    def group(g, carry):
        slot(g * 2, 0, first=False, last=False)
        slot(g * 2 + 1, 1, first=False, last=False)
        return carry

    lax.fori_loop(1, _NBLK // 2 - 1, group, 0, unroll=False)

    slot(_NBLK - 2, 0, first=False, last=False)
    slot(_NBLK - 1, 1, first=False, last=True)

    wait_out(0)
    wait_out(1)


@jax.jit
def kernel(inputs, token_table, pos_table):
    idx = inputs.T.reshape(_SEQ // 8, 8, _NW, _BPW).transpose(0, 2, 1, 3)
    pos = pos_table.reshape(-1)
    tok = jnp.pad(token_table, ((0, 0), (0, 128 - _DIM))).reshape(-1, _DIM)
    run = pl.kernel(
        _body,
        out_type=jax.ShapeDtypeStruct((_SEQ, _DIM // 8, _NW, 8, _BPW), jnp.float32),
        mesh=plsc.VectorSubcoreMesh(core_axis_name="c", subcore_axis_name="s"),
        compiler_params=pltpu.CompilerParams(
            use_tc_tiling_on_sc=False, needs_layout_passes=False),
        scratch_types=[
            pltpu.VMEM((_SEQ // 8, 8, _BPW), jnp.int32),
            pltpu.VMEM((_SEQ * _DIM,), jnp.float32),
            pltpu.VMEM((_NBUF, _SBLK * _BPW, _DIM), jnp.float32),
            pltpu.VMEM((_NBUF, _NT * 8, _PAD), jnp.float32),
            pltpu.SemaphoreType.DMA((_NBUF,)),
            pltpu.SemaphoreType.DMA((_NBUF,)),
        ],
    )
    out5 = run(idx, tok, pos)
    return out5.transpose(2, 4, 0, 1, 3).reshape(_BATCH, _SEQ, _DIM)
